# baseline (speedup 1.0000x reference)
import functools

import jax
import jax.numpy as jnp
from jax import lax
from jax.experimental import pallas as pl
from jax.experimental.pallas import tpu as pltpu
from jax.experimental.pallas import tpu_sc as plsc

EMB_DIM = 32
BATCH = 16384
ROWS_PER_TILE = 128 // EMB_DIM
NC, NS = 2, 16
NW = NC * NS
BPW = BATCH // NW
CHUNK = 128
NCHUNK = BPW // CHUNK
IDX2D = (BATCH // CHUNK, CHUNK)
NUM_TABLE_ROWS = 1000000 // ROWS_PER_TILE

_MESH = plsc.VectorSubcoreMesh(
    core_axis_name="c", subcore_axis_name="s", num_cores=NC, num_subcores=NS)


def _sc_gather_body(ug_hbm, ig_hbm, um_hbm, im_hbm, uid_hbm, iid_hbm,
                    out_ug, out_ig, out_um, out_im,
                    uidx_v, iidx_v, b_ug, b_ig, b_um, b_im, sem):
    wid = lax.axis_index("s") * NC + lax.axis_index("c")
    pltpu.sync_copy(uid_hbm, uidx_v)
    pltpu.sync_copy(iid_hbm, iidx_v)
    for j in range(NCHUNK):
        row = wid * NCHUNK + j
        cs = [pltpu.async_copy(ug_hbm.at[uidx_v.at[row]], b_ug, sem),
              pltpu.async_copy(ig_hbm.at[iidx_v.at[row]], b_ig, sem),
              pltpu.async_copy(um_hbm.at[uidx_v.at[row]], b_um, sem),
              pltpu.async_copy(im_hbm.at[iidx_v.at[row]], b_im, sem)]
        for c in cs:
            c.wait()
        dst = pl.ds(row * CHUNK, CHUNK)
        pltpu.sync_copy(b_ug, out_ug.at[dst])
        pltpu.sync_copy(b_ig, out_ig.at[dst])
        pltpu.sync_copy(b_um, out_um.at[dst])
        pltpu.sync_copy(b_im, out_im.at[dst])


_sc_gather = pl.kernel(
    _sc_gather_body,
    out_type=[jax.ShapeDtypeStruct((BATCH, 128), jnp.float32)] * 4,
    mesh=_MESH,
    scratch_types=[
        pltpu.VMEM(IDX2D, jnp.int32),
        pltpu.VMEM(IDX2D, jnp.int32),
        pltpu.VMEM((CHUNK, 128), jnp.float32),
        pltpu.VMEM((CHUNK, 128), jnp.float32),
        pltpu.VMEM((CHUNK, 128), jnp.float32),
        pltpu.VMEM((CHUNK, 128), jnp.float32),
        pltpu.SemaphoreType.DMA,
    ],
)


def _extract(buf, off):
    r = buf[:, 0:EMB_DIM]
    for j in range(1, ROWS_PER_TILE):
        r = jnp.where(off == j, buf[:, EMB_DIM * j:EMB_DIM * (j + 1)], r)
    return r


def _mlp_body(gu, gi, hu, hi, ou, oi, w1u, w1i, b1, w2t, b2, w3t, b3,
              wfg, wfh, bf, out):
    f32 = jnp.float32
    ou_v, oi_v = ou[...], oi[...]
    um = _extract(hu[...], ou_v)
    im = _extract(hi[...], oi_v)
    h = jnp.dot(um, w1u[...], preferred_element_type=f32)
    h += jnp.dot(im, w1i[...], preferred_element_type=f32)
    h = jnp.maximum(h + b1[...], 0.0)
    h = jnp.maximum(jnp.dot(h, w2t[...], preferred_element_type=f32) + b2[...], 0.0)
    h = jnp.maximum(jnp.dot(h, w3t[...], preferred_element_type=f32) + b3[...], 0.0)
    gmf = _extract(gu[...], ou_v) * _extract(gi[...], oi_v)
    logit = (jnp.dot(gmf, wfg[...], preferred_element_type=f32)
             + jnp.dot(h, wfh[...], preferred_element_type=f32) + bf[...])
    out[...] = jax.nn.sigmoid(logit)


_BS = 2048


def _mlp_call(gu, gi, hu, hi, ou, oi, w1u, w1i, b1, w2t, b2, w3t, b3,
              wfg, wfh, bf):
    row_spec = pl.BlockSpec((_BS, 128), lambda i: (i, 0))
    off_spec = pl.BlockSpec((_BS, 1), lambda i: (i, 0))
    full = pl.BlockSpec(index_map=lambda i: (0, 0))
    return pl.pallas_call(
        _mlp_body,
        grid=(BATCH // _BS,),
        in_specs=[row_spec] * 4 + [off_spec] * 2 + [full] * 10,
        out_specs=pl.BlockSpec((_BS, 1), lambda i: (i, 0)),
        out_shape=jax.ShapeDtypeStruct((BATCH, 1), jnp.float32),
    )(gu, gi, hu, hi, ou, oi, w1u, w1i, b1, w2t, b2, w3t, b3, wfg, wfh, bf)


def kernel(user_emb_gmf, item_emb_gmf, user_emb_mlp, item_emb_mlp,
           W1, b1, W2, b2, W3, b3, Wf, bf, user_ids, item_ids):
    uid = user_ids.astype(jnp.int32)
    iid = item_ids.astype(jnp.int32)
    urow = (uid // ROWS_PER_TILE).reshape(IDX2D)
    irow = (iid // ROWS_PER_TILE).reshape(IDX2D)
    uoff = (uid % ROWS_PER_TILE).reshape(BATCH, 1)
    ioff = (iid % ROWS_PER_TILE).reshape(BATCH, 1)
    wide = (NUM_TABLE_ROWS, 128)
    gu, gi, hu, hi = _sc_gather(
        user_emb_gmf.reshape(wide), item_emb_gmf.reshape(wide),
        user_emb_mlp.reshape(wide), item_emb_mlp.reshape(wide), urow, irow)
    w1u = W1[:, :EMB_DIM].T
    w1i = W1[:, EMB_DIM:].T
    wfg = Wf[:, :EMB_DIM].T
    wfh = Wf[:, EMB_DIM:].T
    return _mlp_call(gu, gi, hu, hi, uoff, ioff, w1u, w1i,
                     b1.reshape(1, -1), W2.T, b2.reshape(1, -1), W3.T,
                     b3.reshape(1, -1), wfg, wfh, bf.reshape(1, 1))


# speedup vs baseline: 3.5402x; 3.5402x over previous
import functools

import jax
import jax.numpy as jnp
from jax import lax
from jax.experimental import pallas as pl
from jax.experimental.pallas import tpu as pltpu
from jax.experimental.pallas import tpu_sc as plsc

EMB_DIM = 32
BATCH = 16384
NC, NS = 2, 16
NW = NC * NS
BPW = BATCH // NW
LANES = 128
GROUPS = BPW // 16
NSLOT = 3
IDX2D = (BATCH // LANES, LANES)

_MESH = plsc.VectorSubcoreMesh(
    core_axis_name="c", subcore_axis_name="s", num_cores=NC, num_subcores=NS)


def _sc_gather_body(ug_hbm, ig_hbm, um_hbm, im_hbm, uid_hbm, iid_hbm,
                    out_ug, out_ig, out_um, out_im,
                    uidx_v, iidx_v, ring_ug, ring_ig, ring_um, ring_im,
                    st_ug, st_ig, st_um, st_im,
                    sem_ug, sem_ig, sem_um, sem_im):
    wid = lax.axis_index("s") * NC + lax.axis_index("c")
    tile0 = pl.multiple_of(8 * (wid // 2), 8)
    pltpu.sync_copy(uid_hbm.at[pl.ds(tile0, 8)], uidx_v)
    pltpu.sync_copy(iid_hbm.at[pl.ds(tile0, 8)], iidx_v)
    row0 = 4 * (wid % 2)
    iota = lax.iota(jnp.int32, 16)
    tabs = ((ug_hbm, ring_ug, st_ug, sem_ug, 0),
            (ig_hbm, ring_ig, st_ig, sem_ig, 1),
            (um_hbm, ring_um, st_um, sem_um, 0),
            (im_hbm, ring_im, st_im, sem_im, 1))

    def group(g, _):
        rvec = (plsc.load_gather(uidx_v, [jnp.full((16,), row0 + g // 8,
                                                   jnp.int32),
                                          iota + 16 * (g % 8)]),
                plsc.load_gather(iidx_v, [jnp.full((16,), row0 + g // 8,
                                                   jnp.int32),
                                          iota + 16 * (g % 8)]))

        def fire(b, slot):
            for hbm, ring, _, sem, which in tabs:
                r = rvec[which][b]
                tcol = pl.multiple_of((r // LANES) * LANES, LANES)
                pltpu.async_copy(hbm.at[:, pl.ds(tcol, LANES)],
                                 ring.at[slot], sem.at[slot])

        def drain_extract(b, slot):
            col = jnp.full((16,), 16 * (g % 8) + b, jnp.int32)
            for hbm, ring, st, sem, which in tabs:
                pltpu.make_async_copy(hbm.at[:, pl.ds(0, LANES)],
                                      ring.at[slot], sem.at[slot]).wait()
                lane = jnp.full((16,), rvec[which][b] % LANES, jnp.int32)
                v0 = plsc.load_gather(ring.at[slot], [iota, lane])
                v1 = plsc.load_gather(ring.at[slot], [iota + 16, lane])
                plsc.store_scatter(st, [col, iota], v0)
                plsc.store_scatter(st, [col, iota + 16], v1)

        for b in range(16):
            slot = b % NSLOT
            if b >= NSLOT:
                drain_extract(b - NSLOT, slot)
            fire(b, slot)
        for b in range(16 - NSLOT, 16):
            drain_extract(b, b % NSLOT)

        @pl.when(g % 8 == 7)
        def _():
            base = pl.multiple_of(BPW * wid + LANES * (g // 8), LANES)
            pltpu.sync_copy(st_ug, out_ug.at[pl.ds(base, LANES)])
            pltpu.sync_copy(st_ig, out_ig.at[pl.ds(base, LANES)])
            pltpu.sync_copy(st_um, out_um.at[pl.ds(base, LANES)])
            pltpu.sync_copy(st_im, out_im.at[pl.ds(base, LANES)])

        return ()

    lax.fori_loop(0, GROUPS, group, (), unroll=False)


_sc_gather = pl.kernel(
    _sc_gather_body,
    out_type=[jax.ShapeDtypeStruct((BATCH, EMB_DIM), jnp.float32)] * 4,
    mesh=_MESH,
    scratch_types=(
        [pltpu.VMEM((8, LANES), jnp.int32)] * 2
        + [pltpu.VMEM((NSLOT, EMB_DIM, LANES), jnp.float32)] * 4
        + [pltpu.VMEM((LANES, EMB_DIM), jnp.float32)] * 4
        + [pltpu.SemaphoreType.DMA((NSLOT,))] * 4
    ),
    compiler_params=pltpu.CompilerParams(needs_layout_passes=False),
)


def _mlp_body(ug, ig, um, im, w1u, w1i, b1, w2t, b2, w3t, b3, wfg, wfh, bf,
              out):
    f32 = jnp.float32
    h = jnp.dot(um[...], w1u[...], preferred_element_type=f32)
    h += jnp.dot(im[...], w1i[...], preferred_element_type=f32)
    h = jnp.maximum(h + b1[...], 0.0)
    h = jnp.maximum(jnp.dot(h, w2t[...], preferred_element_type=f32) + b2[...], 0.0)
    h = jnp.maximum(jnp.dot(h, w3t[...], preferred_element_type=f32) + b3[...], 0.0)
    gmf = ug[...] * ig[...]
    logit = (jnp.dot(gmf, wfg[...], preferred_element_type=f32)
             + jnp.dot(h, wfh[...], preferred_element_type=f32) + bf[...])
    out[...] = jax.nn.sigmoid(logit)


_BS = 2048


def _mlp_call(ug, ig, um, im, w1u, w1i, b1, w2t, b2, w3t, b3, wfg, wfh, bf):
    row_spec = pl.BlockSpec((_BS, EMB_DIM), lambda i: (i, 0))
    full = pl.BlockSpec(index_map=lambda i: (0, 0))
    return pl.pallas_call(
        _mlp_body,
        grid=(BATCH // _BS,),
        in_specs=[row_spec] * 4 + [full] * 10,
        out_specs=pl.BlockSpec((_BS, 1), lambda i: (i, 0)),
        out_shape=jax.ShapeDtypeStruct((BATCH, 1), jnp.float32),
    )(ug, ig, um, im, w1u, w1i, b1, w2t, b2, w3t, b3, wfg, wfh, bf)


def kernel(user_emb_gmf, item_emb_gmf, user_emb_mlp, item_emb_mlp,
           W1, b1, W2, b2, W3, b3, Wf, bf, user_ids, item_ids):
    uid = user_ids.astype(jnp.int32).reshape(IDX2D)
    iid = item_ids.astype(jnp.int32).reshape(IDX2D)
    ug, ig, um, im = _sc_gather(
        user_emb_gmf.T, item_emb_gmf.T, user_emb_mlp.T, item_emb_mlp.T,
        uid, iid)
    w1u = W1[:, :EMB_DIM].T
    w1i = W1[:, EMB_DIM:].T
    wfg = Wf[:, :EMB_DIM].T
    wfh = Wf[:, EMB_DIM:].T
    return _mlp_call(ug, ig, um, im, w1u, w1i, b1.reshape(1, -1),
                     W2.T, b2.reshape(1, -1), W3.T, b3.reshape(1, -1),
                     wfg, wfh, bf.reshape(1, 1))


# speedup vs baseline: 3.7443x; 1.0577x over previous
import functools

import jax
import jax.numpy as jnp
from jax import lax
from jax.experimental import pallas as pl
from jax.experimental.pallas import tpu as pltpu
from jax.experimental.pallas import tpu_sc as plsc

EMB_DIM = 32
BATCH = 16384
NC, NS = 2, 16
NW = NC * NS
BPW = BATCH // NW
LANES = 128
GROUPS = BPW // 16
NSLOT = 4
IDX2D = (BATCH // LANES, LANES)

_MESH = plsc.VectorSubcoreMesh(
    core_axis_name="c", subcore_axis_name="s", num_cores=NC, num_subcores=NS)


def _sc_gather_body(ug_hbm, ig_hbm, um_hbm, im_hbm, uid_hbm, iid_hbm,
                    out_ug, out_ig, out_um, out_im,
                    uidx_v, iidx_v, ring_ug, ring_ig, ring_um, ring_im,
                    st_ug, st_ig, st_um, st_im,
                    sem_ug, sem_ig, sem_um, sem_im):
    wid = lax.axis_index("s") * NC + lax.axis_index("c")
    tile0 = pl.multiple_of(8 * (wid // 2), 8)
    pltpu.sync_copy(uid_hbm.at[pl.ds(tile0, 8)], uidx_v)
    pltpu.sync_copy(iid_hbm.at[pl.ds(tile0, 8)], iidx_v)
    row0 = 4 * (wid % 2)
    iota = lax.iota(jnp.int32, 16)
    tabs = ((ug_hbm, ring_ug, st_ug, sem_ug, 0),
            (ig_hbm, ring_ig, st_ig, sem_ig, 1),
            (um_hbm, ring_um, st_um, sem_um, 0),
            (im_hbm, ring_im, st_im, sem_im, 1))

    def fire(rv, b, slot):
        for hbm, ring, _, sem, which in tabs:
            r = rv[which][b]
            tcol = pl.multiple_of((r // LANES) * LANES, LANES)
            pltpu.async_copy(hbm.at[:, pl.ds(tcol, LANES)],
                             ring.at[slot], sem.at[slot])

    def drain(slot):
        for hbm, ring, _, sem, _w in tabs:
            pltpu.make_async_copy(hbm.at[:, pl.ds(0, LANES)],
                                  ring.at[slot], sem.at[slot]).wait()

    def extract(rv, b, slot, j):
        col = jnp.full((16,), j % 32, jnp.int32)
        buf = (j // 32) % 2
        for hbm, ring, st, sem, which in tabs:
            lane = jnp.full((16,), rv[which][b] % LANES, jnp.int32)
            v0 = plsc.load_gather(ring.at[slot], [iota, lane])
            v1 = plsc.load_gather(ring.at[slot], [iota + 16, lane])
            plsc.store_scatter(st.at[buf], [col, iota], v0)
            plsc.store_scatter(st.at[buf], [col, iota + 16], v1)

    def writeback(block):
        buf = block % 2
        base = pl.multiple_of(BPW * wid + 32 * block, 32)
        pltpu.sync_copy(st_ug.at[buf], out_ug.at[pl.ds(base, 32)])
        pltpu.sync_copy(st_ig.at[buf], out_ig.at[pl.ds(base, 32)])
        pltpu.sync_copy(st_um.at[buf], out_um.at[pl.ds(base, 32)])
        pltpu.sync_copy(st_im.at[buf], out_im.at[pl.ds(base, 32)])

    def group(g, carry):
        rcur = (plsc.load_gather(uidx_v, [jnp.full((16,), row0 + g // 8,
                                                   jnp.int32),
                                          iota + 16 * (g % 8)]),
                plsc.load_gather(iidx_v, [jnp.full((16,), row0 + g // 8,
                                                   jnp.int32),
                                          iota + 16 * (g % 8)]))
        for b in range(NSLOT):
            @pl.when(g != 0)
            def _(b=b):
                drain(b)
                extract(carry, 12 + b, b, 16 * g + b - 4)
            fire(rcur, b, b)
        for b in range(NSLOT, 16):
            slot = b % NSLOT
            drain(slot)
            extract(rcur, b - 4, slot, 16 * g + b - 4)
            fire(rcur, b, slot)
        @pl.when(jnp.logical_and(g % 2 == 0, g >= 2))
        def _():
            writeback(g // 2 - 1)
        return rcur

    rlast = lax.fori_loop(0, GROUPS, group, (jnp.zeros((16,), jnp.int32),
                                             jnp.zeros((16,), jnp.int32)),
                          unroll=False)
    for b in range(NSLOT):
        drain(b)
        extract(rlast, 12 + b, b, BPW - 4 + b)
    writeback(15)


_sc_gather = pl.kernel(
    _sc_gather_body,
    out_type=[jax.ShapeDtypeStruct((BATCH, EMB_DIM), jnp.float32)] * 4,
    mesh=_MESH,
    scratch_types=(
        [pltpu.VMEM((8, LANES), jnp.int32)] * 2
        + [pltpu.VMEM((NSLOT, EMB_DIM, LANES), jnp.float32)] * 4
        + [pltpu.VMEM((2, 32, EMB_DIM), jnp.float32)] * 4
        + [pltpu.SemaphoreType.DMA((NSLOT,))] * 4
    ),
    compiler_params=pltpu.CompilerParams(needs_layout_passes=False),
)


def _mlp_body(ug, ig, um, im, w1u, w1i, b1, w2t, b2, w3t, b3, wfg, wfh, bf,
              out):
    f32 = jnp.float32
    h = jnp.dot(um[...], w1u[...], preferred_element_type=f32)
    h += jnp.dot(im[...], w1i[...], preferred_element_type=f32)
    h = jnp.maximum(h + b1[...], 0.0)
    h = jnp.maximum(jnp.dot(h, w2t[...], preferred_element_type=f32) + b2[...], 0.0)
    h = jnp.maximum(jnp.dot(h, w3t[...], preferred_element_type=f32) + b3[...], 0.0)
    gmf = ug[...] * ig[...]
    logit = (jnp.dot(gmf, wfg[...], preferred_element_type=f32)
             + jnp.dot(h, wfh[...], preferred_element_type=f32) + bf[...])
    out[...] = jax.nn.sigmoid(logit)


_BS = 2048


def _mlp_call(ug, ig, um, im, w1u, w1i, b1, w2t, b2, w3t, b3, wfg, wfh, bf):
    row_spec = pl.BlockSpec((_BS, EMB_DIM), lambda i: (i, 0))
    full = pl.BlockSpec(index_map=lambda i: (0, 0))
    return pl.pallas_call(
        _mlp_body,
        grid=(BATCH // _BS,),
        in_specs=[row_spec] * 4 + [full] * 10,
        out_specs=pl.BlockSpec((_BS, 1), lambda i: (i, 0)),
        out_shape=jax.ShapeDtypeStruct((BATCH, 1), jnp.float32),
    )(ug, ig, um, im, w1u, w1i, b1, w2t, b2, w3t, b3, wfg, wfh, bf)


def kernel(user_emb_gmf, item_emb_gmf, user_emb_mlp, item_emb_mlp,
           W1, b1, W2, b2, W3, b3, Wf, bf, user_ids, item_ids):
    uid = user_ids.astype(jnp.int32).reshape(IDX2D)
    iid = item_ids.astype(jnp.int32).reshape(IDX2D)
    ug, ig, um, im = _sc_gather(
        user_emb_gmf.T, item_emb_gmf.T, user_emb_mlp.T, item_emb_mlp.T,
        uid, iid)
    w1u = W1[:, :EMB_DIM].T
    w1i = W1[:, EMB_DIM:].T
    wfg = Wf[:, :EMB_DIM].T
    wfh = Wf[:, EMB_DIM:].T
    return _mlp_call(ug, ig, um, im, w1u, w1i, b1.reshape(1, -1),
                     W2.T, b2.reshape(1, -1), W3.T, b3.reshape(1, -1),
                     wfg, wfh, bf.reshape(1, 1))
